# baseline (speedup 1.0000x reference)
import functools

import jax
import jax.numpy as jnp
from jax import lax
from jax.experimental import pallas as pl
from jax.experimental.pallas import tpu as pltpu
from jax.experimental.pallas import tpu_sc as plsc

_VOCAB = 1000000
_D = 64
_B = 16384
_W = 16
_TOT = _B * _W
_NC = 2
_NS = 16
_NW = _NC * _NS
_PER_W = _TOT // _NW
_CHUNK = 256
_NCH = _PER_W // _CHUNK
_NBUF = 4
assert _NCH % _NBUF == 0

_mesh = plsc.VectorSubcoreMesh(core_axis_name="c", subcore_axis_name="s")


@functools.partial(
    pl.kernel,
    mesh=_mesh,
    out_type=jax.ShapeDtypeStruct((_TOT, _D), jnp.float32),
    compiler_params=pltpu.CompilerParams(use_tc_tiling_on_sc=False),
    scratch_types=[
        pltpu.VMEM((_NCH, _CHUNK), jnp.int32),
        pltpu.VMEM((_NBUF, _CHUNK, _D), jnp.float32),
        [pltpu.SemaphoreType.DMA] * _NBUF,
        [pltpu.SemaphoreType.DMA] * _NBUF,
    ],
)
def _gather_kernel(idx_hbm, table_hbm, out_hbm, idx_v, rows_v, gsems, ssems):
    wid = lax.axis_index("s") * _NC + lax.axis_index("c")
    base = wid * _PER_W
    pltpu.sync_copy(idx_hbm.at[wid], idx_v)

    def gather(c, b):
        return pltpu.make_async_copy(
            table_hbm.at[idx_v.at[c]], rows_v.at[b], gsems[b])

    def store(c, b):
        return pltpu.make_async_copy(
            rows_v.at[b], out_hbm.at[pl.ds(base + c * _CHUNK, _CHUNK)],
            ssems[b])

    for b in range(_NBUF - 1):
        gather(b, b).start()

    def group(g, carry):
        c0 = g * _NBUF
        for b in range(_NBUF):
            c = c0 + b
            nc = c + _NBUF - 1
            nb = (b + _NBUF - 1) % _NBUF

            @pl.when(nc < _NCH)
            def _():
                @pl.when(c >= 1)
                def _():
                    store(c - 1, nb).wait()
                gather(nc, nb).start()

            gather(c, b).wait()
            store(c, b).start()
        return carry

    lax.fori_loop(0, _NCH // _NBUF, group, 0)
    for b in range(_NBUF):
        store(0, b).wait()


def kernel(sentence, word_embd):
    idx = sentence.astype(jnp.int32).reshape(_NW, _NCH, _CHUNK)
    out = _gather_kernel(idx, word_embd)
    return out.reshape(_B, _W, _D)


# speedup vs baseline: 1.1638x; 1.1638x over previous
import functools

import jax
import jax.numpy as jnp
from jax import lax
from jax.experimental import pallas as pl
from jax.experimental.pallas import tpu as pltpu
from jax.experimental.pallas import tpu_sc as plsc

_VOCAB = 1000000
_D = 64
_DP = 128
_B = 16384
_W = 16
_TOT = _B * _W
_NC = 2
_NS = 16
_NW = _NC * _NS
_PER_W = _TOT // _NW
_CHUNK = 128
_NCH = _PER_W // _CHUNK
_NBUF = 4
assert _NCH % _NBUF == 0

_mesh = plsc.VectorSubcoreMesh(core_axis_name="c", subcore_axis_name="s")


@functools.partial(
    pl.kernel,
    mesh=_mesh,
    out_type=jax.ShapeDtypeStruct((_TOT, _DP), jnp.float32),
    compiler_params=pltpu.CompilerParams(use_tc_tiling_on_sc=True),
    scratch_types=[
        pltpu.VMEM((_NCH, _CHUNK), jnp.int32),
        pltpu.VMEM((_NBUF, _CHUNK, _DP), jnp.float32),
        [pltpu.SemaphoreType.DMA] * _NBUF,
        [pltpu.SemaphoreType.DMA] * _NBUF,
    ],
)
def _gather_kernel(idx_hbm, table_hbm, out_hbm, idx_v, rows_v, gsems, ssems):
    wid = lax.axis_index("s") * _NC + lax.axis_index("c")
    base = wid * _PER_W
    pltpu.sync_copy(idx_hbm.at[wid], idx_v)

    def gather(c, b):
        return pltpu.make_async_copy(
            table_hbm.at[idx_v.at[c]], rows_v.at[b], gsems[b])

    def store(c, b):
        return pltpu.make_async_copy(
            rows_v.at[b], out_hbm.at[pl.ds(base + c * _CHUNK, _CHUNK)],
            ssems[b])

    for b in range(_NBUF - 1):
        gather(b, b).start()

    def group(g, carry):
        c0 = g * _NBUF
        for b in range(_NBUF):
            c = c0 + b
            nc = c + _NBUF - 1
            nb = (b + _NBUF - 1) % _NBUF

            @pl.when(nc < _NCH)
            def _():
                @pl.when(c >= 1)
                def _():
                    store(c - 1, nb).wait()
                gather(nc, nb).start()

            gather(c, b).wait()
            store(c, b).start()
        return carry

    lax.fori_loop(0, _NCH // _NBUF, group, 0)
    for b in range(_NBUF):
        store(0, b).wait()


def kernel(sentence, word_embd):
    tpad = jnp.pad(word_embd, ((0, 0), (0, _DP - _D)))
    idx = sentence.astype(jnp.int32).reshape(_NW, _NCH, _CHUNK)
    out = _gather_kernel(idx, tpad)
    return out[:, :_D].reshape(_B, _W, _D)


# speedup vs baseline: 1.4464x; 1.2429x over previous
import functools

import jax
import jax.numpy as jnp
from jax import lax
from jax.experimental import pallas as pl
from jax.experimental.pallas import tpu as pltpu
from jax.experimental.pallas import tpu_sc as plsc

_VOCAB = 1000000
_D = 64
_B = 16384
_W = 16
_TOT = _B * _W
_NC = 2
_NS = 16
_NW = _NC * _NS
_PER_W = _TOT // _NW
_CHUNK = 128
_NCH = _PER_W // _CHUNK

_mesh = plsc.VectorSubcoreMesh(core_axis_name="c", subcore_axis_name="s")


@functools.partial(
    pl.kernel,
    mesh=_mesh,
    out_type=jax.ShapeDtypeStruct((_TOT, _D), jnp.float32),
    compiler_params=pltpu.CompilerParams(use_tc_tiling_on_sc=True),
    scratch_types=[
        pltpu.VMEM((_NCH, _CHUNK), jnp.int32),
        pltpu.VMEM((2, _CHUNK, _D), jnp.float32),
        pltpu.SemaphoreType.DMA,
        pltpu.SemaphoreType.DMA,
        pltpu.SemaphoreType.DMA,
    ],
)
def _gather_kernel(idx_hbm, table_hbm, out_hbm, idx_v, rows_v, isem, gsem, ssem):
    wid = lax.axis_index("s") * _NC + lax.axis_index("c")
    base = wid * _PER_W

    pltpu.sync_copy(idx_hbm.at[wid], idx_v)

    def chunk_body(c, carry):
        def grp_body(g, carry2):
            vec = idx_v[c, pl.ds(g * 16, 16)]
            for i in range(16):
                pltpu.make_async_copy(
                    table_hbm.at[pl.ds(vec[i], 1)],
                    rows_v.at[0, pl.ds(g * 16 + i, 1)],
                    gsem,
                ).start()
            return carry2

        lax.fori_loop(0, _CHUNK // 16, grp_body, 0)
        pltpu.make_async_copy(
            table_hbm.at[pl.ds(0, _CHUNK)], rows_v.at[0], gsem).wait()
        pltpu.sync_copy(
            rows_v.at[0], out_hbm.at[pl.ds(base + c * _CHUNK, _CHUNK)])
        return carry

    lax.fori_loop(0, _NCH, chunk_body, 0)


def kernel(sentence, word_embd):
    idx = sentence.astype(jnp.int32).reshape(_NW, _NCH, _CHUNK)
    out = _gather_kernel(idx, word_embd)
    return out.reshape(_B, _W, _D)


# speedup vs baseline: 1.5812x; 1.0931x over previous
import functools

import jax
import jax.numpy as jnp
from jax import lax
from jax.experimental import pallas as pl
from jax.experimental.pallas import tpu as pltpu
from jax.experimental.pallas import tpu_sc as plsc

_VOCAB = 1000000
_D = 64
_B = 16384
_W = 16
_TOT = _B * _W
_NC = 2
_NS = 16
_NW = _NC * _NS
_PER_W = _TOT // _NW
_CHUNK = 128
_NCH = _PER_W // _CHUNK
assert _NCH % 2 == 0

_mesh = plsc.VectorSubcoreMesh(core_axis_name="c", subcore_axis_name="s")


@functools.partial(
    pl.kernel,
    mesh=_mesh,
    out_type=jax.ShapeDtypeStruct((_TOT, _D), jnp.float32),
    compiler_params=pltpu.CompilerParams(use_tc_tiling_on_sc=True),
    scratch_types=[
        pltpu.VMEM((_NCH, _CHUNK), jnp.int32),
        pltpu.VMEM((2, _CHUNK, _D), jnp.float32),
        [pltpu.SemaphoreType.DMA] * 2,
        [pltpu.SemaphoreType.DMA] * 2,
    ],
)
def _gather_kernel(idx_hbm, table_hbm, out_hbm, idx_v, rows_v, gsems, ssems):
    wid = lax.axis_index("s") * _NC + lax.axis_index("c")
    base = wid * _PER_W
    pltpu.sync_copy(idx_hbm.at[wid], idx_v)

    def issue(c, slot):
        def grp_body(g, carry):
            vec = idx_v[c, pl.ds(g * 16, 16)]
            for i in range(16):
                pltpu.make_async_copy(
                    table_hbm.at[pl.ds(vec[i], 1)],
                    rows_v.at[slot, pl.ds(g * 16 + i, 1)],
                    gsems[slot],
                ).start()
            return carry

        lax.fori_loop(0, _CHUNK // 16, grp_body, 0)

    def drain(slot):
        pltpu.make_async_copy(
            table_hbm.at[pl.ds(0, _CHUNK)], rows_v.at[slot], gsems[slot]
        ).wait()

    def store_start(c, slot):
        pltpu.make_async_copy(
            rows_v.at[slot],
            out_hbm.at[pl.ds(base + c * _CHUNK, _CHUNK)],
            ssems[slot],
        ).start()

    def store_wait(slot):
        pltpu.make_async_copy(
            rows_v.at[slot], out_hbm.at[pl.ds(base, _CHUNK)], ssems[slot]
        ).wait()

    issue(0, 0)

    def pair_body(p, carry):
        c0 = 2 * p

        @pl.when(p >= 1)
        def _():
            store_wait(1)
        issue(c0 + 1, 1)
        drain(0)
        store_start(c0, 0)

        @pl.when(c0 + 2 < _NCH)
        def _():
            store_wait(0)
            issue(c0 + 2, 0)
        drain(1)
        store_start(c0 + 1, 1)
        return carry

    lax.fori_loop(0, _NCH // 2, pair_body, 0)
    store_wait(0)
    store_wait(1)


def kernel(sentence, word_embd):
    idx = sentence.astype(jnp.int32).reshape(_NW, _NCH, _CHUNK)
    out = _gather_kernel(idx, word_embd)
    return out.reshape(_B, _W, _D)
